# baseline (speedup 1.0000x reference)
import functools

import jax
import jax.numpy as jnp
from jax import lax
from jax.experimental import pallas as pl
from jax.experimental.pallas import tpu as pltpu
from jax.experimental.pallas import tpu_sc as plsc

_B, _S, _D = 4, 8192, 1024
_SC_BATCHES = (3,)
_TC_NB = _B - len(_SC_BATCHES)


_BS = 2048


def _tc_body(x_ref, pos_ref, o_ref):
    o_ref[...] = x_ref[...] + pos_ref[...][None, :, :]


def _kernel_tc(x, pos_table):
    n_s = _S // _BS
    return pl.pallas_call(
        _tc_body,
        grid=(n_s, _TC_NB),
        in_specs=[
            pl.BlockSpec((1, _BS, _D), lambda s, b: (b, s, 0)),
            pl.BlockSpec((_BS, _D), lambda s, b: (s, 0)),
        ],
        out_specs=pl.BlockSpec((1, _BS, _D), lambda s, b: (b, s, 0)),
        out_shape=jax.ShapeDtypeStruct((_TC_NB, _S, _D), x.dtype),
    )(x, pos_table)


_NC, _NS = 2, 16
_NW = _NC * _NS
_ROWS_PER_W = _S // _NW
_SB = 16
_NSUB = _ROWS_PER_W // _SB
_NXB = 4
_LOOK = 2
_UNROLL = 8


def _add_block(xb, pb):
    n_chunks_per_row = _D // 16

    def body(j, carry):
        base = j * _UNROLL
        for u in range(_UNROLL):
            idx = base + u
            r = idx // n_chunks_per_row
            c = lax.rem(idx, n_chunks_per_row) * 16
            sl = pl.ds(c, 16)
            xb[r, sl] = xb[r, sl] + pb[r, sl]
        return carry

    lax.fori_loop(0, (_SB * n_chunks_per_row) // _UNROLL, body, 0)


def _sc_kernel_body(x_hbm, pos_hbm, out_hbm, xb0, xb1, xb2, xb3, pb,
                    ls0, ls1, ls2, ls3, ss0, ss1, ss2, ss3):
    xbufs = (xb0, xb1, xb2, xb3)
    load_sems = (ls0, ls1, ls2, ls3)
    store_sems = (ss0, ss1, ss2, ss3)
    wid = lax.axis_index("s") * _NC + lax.axis_index("c")
    row0 = wid * _ROWS_PER_W

    nb = len(_SC_BATCHES)
    nt = _NSUB * nb
    loads = [None] * nt
    stores = [None] * nt

    def rows(t):
        sub, bi = divmod(t, nb)
        return _SC_BATCHES[bi], bi, pl.ds(row0 + sub * _SB, _SB)

    def start_load(t):
        k = t % _NXB
        b, _, sl = rows(t)
        loads[t] = pltpu.async_copy(x_hbm.at[b, sl], xbufs[k], load_sems[k])

    for t in range(min(_LOOK, nt)):
        start_load(t)

    for t in range(nt):
        k = t % _NXB
        sub, bi = divmod(t, nb)
        if bi == 0:
            pltpu.sync_copy(pos_hbm.at[pl.ds(row0 + sub * _SB, _SB)], pb)
        loads[t].wait()
        _add_block(xbufs[k], pb)
        _, bo, sl = rows(t)
        stores[t] = pltpu.async_copy(xbufs[k], out_hbm.at[bo, sl],
                                     store_sems[k])
        nxt = t + _LOOK
        if nxt < nt:
            prev = nxt - _NXB
            if prev >= 0:
                stores[prev].wait()
            start_load(nxt)

    for t in range(max(0, nt - _NXB), nt):
        stores[t].wait()


def _kernel_sc(x, pos_table):
    mesh = plsc.VectorSubcoreMesh(core_axis_name="c", subcore_axis_name="s")
    run = functools.partial(
        pl.kernel,
        mesh=mesh,
        out_type=jax.ShapeDtypeStruct((len(_SC_BATCHES), _S, _D), jnp.float32),
        scratch_types=(
            [pltpu.VMEM((_SB, _D), jnp.float32)] * (_NXB + 1)
            + [pltpu.SemaphoreType.DMA] * (2 * _NXB)
        ),
        compiler_params=pltpu.CompilerParams(use_tc_tiling_on_sc=True),
    )(_sc_kernel_body)
    return run(x, pos_table)


def kernel(x, pos_table):
    tc_out = _kernel_tc(x, pos_table)
    sc_out = _kernel_sc(x, pos_table)
    return jnp.concatenate([tc_out, sc_out], axis=0)


# speedup vs baseline: 2.1260x; 2.1260x over previous
import jax
import jax.numpy as jnp
from jax.experimental import pallas as pl

_BS = 1024


def _add_body(x_ref, pos_ref, o_ref):
    o_ref[...] = x_ref[...] + pos_ref[...][None, :, :]


def kernel(x, pos_table):
    B, S, D = x.shape
    n_s = S // _BS
    return pl.pallas_call(
        _add_body,
        grid=(n_s, B),
        in_specs=[
            pl.BlockSpec((1, _BS, D), lambda s, b: (b, s, 0)),
            pl.BlockSpec((_BS, D), lambda s, b: (s, 0)),
        ],
        out_specs=pl.BlockSpec((1, _BS, D), lambda s, b: (b, s, 0)),
        out_shape=jax.ShapeDtypeStruct((B, S, D), x.dtype),
    )(x, pos_table)
